# baseline (speedup 1.0000x reference)
import functools

import jax
import jax.numpy as jnp
from jax.experimental import pallas as pl
from jax.experimental.pallas import tpu as pltpu

N = 4096
D = 256
K = 16
INV_T = 10.0
BLOCK = 512
NEG = -3.0
BIG = 4.0
LOG2E = 1.4426950408889634
TW = 128
S = N // TW


def _norm_body(x_ref, h_ref, l_ref):
    x = x_ref[...]
    n = jnp.maximum(jnp.sqrt(jnp.sum(x * x, axis=-1, keepdims=True)), 1e-12)
    xn = x / n
    h = xn.astype(jnp.bfloat16)
    h_ref[...] = h
    l_ref[...] = (xn - h.astype(jnp.float32)).astype(jnp.bfloat16)


def _body(xbh_ref, xbl_ref, xfh_ref, xfl_ref, o_ref):
    xbh = xbh_ref[...]
    xbl = xbl_ref[...]

    neg = jnp.full((BLOCK, TW), NEG, jnp.float32)
    a1, a2, a3, a4, a5 = neg, neg, neg, neg, neg
    sims = []
    for g in range(S):
        th = xfh_ref[g * TW:(g + 1) * TW, :]
        tl = xfl_ref[g * TW:(g + 1) * TW, :]
        dn = (((1,), (1,)), ((), ()))
        v = (
            jax.lax.dot_general(xbh, th, dn, preferred_element_type=jnp.float32)
            + jax.lax.dot_general(xbh, tl, dn, preferred_element_type=jnp.float32)
            + jax.lax.dot_general(xbl, th, dn, preferred_element_type=jnp.float32)
        )
        sims.append(v)
        lo = jnp.minimum(a1, v)
        a1 = jnp.maximum(a1, v)
        lo2 = jnp.minimum(a2, lo)
        a2 = jnp.maximum(a2, lo)
        lo3 = jnp.minimum(a3, lo2)
        a3 = jnp.maximum(a3, lo2)
        lo4 = jnp.minimum(a4, lo3)
        a4 = jnp.maximum(a4, lo3)
        a5 = jnp.maximum(a5, lo4)

    cand = jnp.concatenate([a1, a2, a3, a4], axis=-1)

    m0 = jnp.max(a1, axis=-1, keepdims=True)
    w = cand
    t = m0
    for _ in range(K - 1):
        w = jnp.where(w >= t, NEG, w)
        t = jnp.max(w, axis=-1, keepdims=True)

    e_cand = jnp.where(cand >= t, jnp.exp((cand - m0) * INV_T), 0.0)
    s0 = jnp.sum(e_cand, axis=-1, keepdims=True)
    t_init = t
    s_init = s0

    def _fixed_ts():
        cacc = functools.reduce(
            jnp.add, [jnp.where(sg >= t, 1.0, 0.0) for sg in sims]
        )
        count = jnp.sum(cacc, axis=-1, keepdims=True)

        def _cond(state):
            _, count_, it = state
            return jnp.logical_and(jnp.any(count_ > float(K)), it < 24)

        def _repair(state):
            t_, count_, it = state
            bad = count_ > float(K)
            macc = functools.reduce(
                jnp.minimum,
                [jnp.where(sg >= t_, sg, BIG) for sg in sims],
            )
            m = jnp.min(macc, axis=-1, keepdims=True)
            nacc = functools.reduce(
                jnp.minimum,
                [jnp.where(sg > m, sg, BIG) for sg in sims],
            )
            tn = jnp.min(nacc, axis=-1, keepdims=True)
            t2_ = jnp.where(jnp.logical_and(bad, tn < BIG), tn, t_)
            c2acc = functools.reduce(
                jnp.add, [jnp.where(sg >= t2_, 1.0, 0.0) for sg in sims]
            )
            c2 = jnp.sum(c2acc, axis=-1, keepdims=True)
            return t2_, c2, it + 1

        t2, _, _ = jax.lax.while_loop(_cond, _repair, (t, count, 0))
        eacc = functools.reduce(
            jnp.add,
            [
                jnp.where(sg >= t2, jnp.exp((sg - m0) * INV_T), 0.0)
                for sg in sims
            ],
        )
        s2 = jnp.sum(eacc, axis=-1, keepdims=True)
        return t2, s2

    t, s = jax.lax.cond(
        jnp.any(a5 >= t),
        _fixed_ts,
        lambda: (t_init, s_init),
    )

    c1 = INV_T * LOG2E
    bias = m0 * c1 + jnp.log2(s)
    for g in range(S):
        sg = sims[g]
        o_ref[:, g * TW:(g + 1) * TW] = jnp.where(
            sg >= t, jnp.exp2(sg * c1 - bias), 0.0
        )


def kernel(X_c):
    Xh, Xl = pl.pallas_call(
        _norm_body,
        grid=(4,),
        in_specs=[pl.BlockSpec((N // 4, D), lambda i: (i, 0))],
        out_specs=[
            pl.BlockSpec((N // 4, D), lambda i: (i, 0)),
            pl.BlockSpec((N // 4, D), lambda i: (i, 0)),
        ],
        out_shape=[
            jax.ShapeDtypeStruct((N, D), jnp.bfloat16),
            jax.ShapeDtypeStruct((N, D), jnp.bfloat16),
        ],
    )(X_c)
    return pl.pallas_call(
        _body,
        grid=(N // BLOCK,),
        in_specs=[
            pl.BlockSpec((BLOCK, D), lambda i: (i, 0)),
            pl.BlockSpec((BLOCK, D), lambda i: (i, 0)),
            pl.BlockSpec((N, D), lambda i: (0, 0)),
            pl.BlockSpec((N, D), lambda i: (0, 0)),
        ],
        out_specs=pl.BlockSpec((BLOCK, N), lambda i: (i, 0)),
        out_shape=jax.ShapeDtypeStruct((N, N), jnp.float32),
        compiler_params=pltpu.CompilerParams(
            dimension_semantics=("arbitrary",),
        ),
    )(Xh, Xl, Xh, Xl)


# speedup vs baseline: 1.4189x; 1.4189x over previous
import functools

import jax
import jax.numpy as jnp
from jax.experimental import pallas as pl
from jax.experimental.pallas import tpu as pltpu

N = 4096
D = 256
K = 16
INV_T = 10.0
BLOCK = 512
NEG = -3.0
BIG = 4.0
LOG2E = 1.4426950408889634
TW = 128
S = N // TW


def _norm_body(x_ref, o_ref):
    x = x_ref[...]
    n = jnp.maximum(jnp.sqrt(jnp.sum(x * x, axis=-1, keepdims=True)), 1e-12)
    o_ref[...] = x / n


def _body(xb_ref, xf_ref, o_ref):
    xb = xb_ref[...]

    neg = jnp.full((BLOCK, TW), NEG, jnp.float32)
    a1, a2, a3, a4, a5 = neg, neg, neg, neg, neg
    sims = []
    for g in range(S):
        v = jax.lax.dot_general(
            xb,
            xf_ref[g * TW:(g + 1) * TW, :],
            (((1,), (1,)), ((), ())),
            preferred_element_type=jnp.float32,
        )
        sims.append(v)
        lo = jnp.minimum(a1, v)
        a1 = jnp.maximum(a1, v)
        lo2 = jnp.minimum(a2, lo)
        a2 = jnp.maximum(a2, lo)
        lo3 = jnp.minimum(a3, lo2)
        a3 = jnp.maximum(a3, lo2)
        lo4 = jnp.minimum(a4, lo3)
        a4 = jnp.maximum(a4, lo3)
        a5 = jnp.maximum(a5, lo4)

    cand = jnp.concatenate([a1, a2, a3, a4], axis=-1)

    m0 = jnp.max(a1, axis=-1, keepdims=True)
    w = cand
    t = m0
    for _ in range(K - 1):
        w = jnp.where(w >= t, NEG, w)
        t = jnp.max(w, axis=-1, keepdims=True)

    e_cand = jnp.where(cand >= t, jnp.exp((cand - m0) * INV_T), 0.0)
    s0 = jnp.sum(e_cand, axis=-1, keepdims=True)
    t_init = t
    s_init = s0

    def _fixed_ts():
        cacc = functools.reduce(
            jnp.add, [jnp.where(sg >= t, 1.0, 0.0) for sg in sims]
        )
        count = jnp.sum(cacc, axis=-1, keepdims=True)

        def _cond(state):
            _, count_, it = state
            return jnp.logical_and(jnp.any(count_ > float(K)), it < 24)

        def _repair(state):
            t_, count_, it = state
            bad = count_ > float(K)
            macc = functools.reduce(
                jnp.minimum,
                [jnp.where(sg >= t_, sg, BIG) for sg in sims],
            )
            m = jnp.min(macc, axis=-1, keepdims=True)
            nacc = functools.reduce(
                jnp.minimum,
                [jnp.where(sg > m, sg, BIG) for sg in sims],
            )
            tn = jnp.min(nacc, axis=-1, keepdims=True)
            t2_ = jnp.where(jnp.logical_and(bad, tn < BIG), tn, t_)
            c2acc = functools.reduce(
                jnp.add, [jnp.where(sg >= t2_, 1.0, 0.0) for sg in sims]
            )
            c2 = jnp.sum(c2acc, axis=-1, keepdims=True)
            return t2_, c2, it + 1

        t2, _, _ = jax.lax.while_loop(_cond, _repair, (t, count, 0))
        eacc = functools.reduce(
            jnp.add,
            [
                jnp.where(sg >= t2, jnp.exp((sg - m0) * INV_T), 0.0)
                for sg in sims
            ],
        )
        s2 = jnp.sum(eacc, axis=-1, keepdims=True)
        return t2, s2

    t, s = jax.lax.cond(
        jnp.any(a5 >= t),
        _fixed_ts,
        lambda: (t_init, s_init),
    )

    c1 = INV_T * LOG2E
    bias = m0 * c1 + jnp.log2(s)
    for g in range(S):
        sg = sims[g]
        o_ref[:, g * TW:(g + 1) * TW] = jnp.where(
            sg >= t, jnp.exp2(sg * c1 - bias), 0.0
        )


def kernel(X_c):
    Xn = pl.pallas_call(
        _norm_body,
        grid=(4,),
        in_specs=[pl.BlockSpec((N // 4, D), lambda i: (i, 0))],
        out_specs=pl.BlockSpec((N // 4, D), lambda i: (i, 0)),
        out_shape=jax.ShapeDtypeStruct((N, D), jnp.float32),
    )(X_c)
    return pl.pallas_call(
        _body,
        grid=(N // BLOCK,),
        in_specs=[
            pl.BlockSpec((BLOCK, D), lambda i: (i, 0)),
            pl.BlockSpec((N, D), lambda i: (0, 0)),
        ],
        out_specs=pl.BlockSpec((BLOCK, N), lambda i: (i, 0)),
        out_shape=jax.ShapeDtypeStruct((N, N), jnp.float32),
        compiler_params=pltpu.CompilerParams(
            dimension_semantics=("parallel",),
        ),
    )(Xn, Xn)
